# baseline (speedup 1.0000x reference)
import functools
import jax
import jax.numpy as jnp
from jax import lax
from jax.experimental import pallas as pl
from jax.experimental.pallas import tpu as pltpu
from jax.experimental.pallas import tpu_sc as plsc

BATCH = 16384
EMBED_DIM = 64

_NC = 2
_NS = 16
_L = 16
_NW = _NC * _NS
_BPW = BATCH // _NW
_CHUNK = 128
_NCHUNK = _BPW // _CHUNK


def _sc_body(uids, iids, utab, itab, out, uidx, iidx, urows, irows, outv, sem):
    wid = lax.axis_index("s") * _NC + lax.axis_index("c")
    base = wid * _BPW

    for j in range(_NCHUNK):
        pltpu.sync_copy(uids.at[pl.ds(base + j * _CHUNK, _CHUNK)], uidx.at[j])
        pltpu.sync_copy(iids.at[pl.ds(base + j * _CHUNK, _CHUNK)], iidx.at[j])

    one = jnp.ones((_L,), jnp.int32)
    for j in range(_NCHUNK):
        for i in range(_CHUNK // _L):
            sl = pl.ds(i * _L, _L)
            uidx[j, sl] = uidx[j, sl] + one
            iidx[j, sl] = iidx[j, sl] + one

    copies = []
    for j in range(_NCHUNK):
        rs = pl.ds(j * _CHUNK, _CHUNK)
        copies.append(pltpu.async_copy(utab.at[uidx.at[j]], urows.at[rs], sem))
        copies.append(pltpu.async_copy(itab.at[iidx.at[j]], irows.at[rs], sem))
    for c in copies:
        c.wait()

    lanes = lax.iota(jnp.int32, _L)

    def group(g, _):
        rows = g * _L + lanes

        def dstep(d, acc):
            col = jnp.full((_L,), d, jnp.int32)
            u = plsc.load_gather(urows, [rows, col])
            v = plsc.load_gather(irows, [rows, col])
            return acc + u * v

        acc = lax.fori_loop(0, EMBED_DIM, dstep,
                            jnp.zeros((_L,), jnp.float32))
        outv[pl.ds(g * _L, _L)] = acc
        return 0

    lax.fori_loop(0, _BPW // _L, group, 0)

    pltpu.sync_copy(outv, out.at[pl.ds(base, _BPW)])


@jax.jit
def kernel(user_ids, item_ids, user_table, item_table):
    mesh = plsc.VectorSubcoreMesh(
        core_axis_name="c", subcore_axis_name="s",
        num_cores=_NC, num_subcores=_NS)
    run = pl.kernel(
        _sc_body,
        out_type=jax.ShapeDtypeStruct((BATCH,), jnp.float32),
        mesh=mesh,
        scratch_types=[
            pltpu.VMEM((_NCHUNK, _CHUNK), jnp.int32),
            pltpu.VMEM((_NCHUNK, _CHUNK), jnp.int32),
            pltpu.VMEM((_BPW, EMBED_DIM), jnp.float32),
            pltpu.VMEM((_BPW, EMBED_DIM), jnp.float32),
            pltpu.VMEM((_BPW,), jnp.float32),
            pltpu.SemaphoreType.DMA,
        ],
        compiler_params=pltpu.CompilerParams(
            needs_layout_passes=False, use_tc_tiling_on_sc=False),
    )
    return run(user_ids, item_ids, user_table, item_table)


# speedup vs baseline: 1.5610x; 1.5610x over previous
import functools
import jax
import jax.numpy as jnp
from jax import lax
from jax.experimental import pallas as pl
from jax.experimental.pallas import tpu as pltpu
from jax.experimental.pallas import tpu_sc as plsc

BATCH = 16384
EMBED_DIM = 64

_NC = 2
_NS = 16
_L = 16
_NW = _NC * _NS
_BPW = BATCH // _NW
_CH = 256
_NCH = _BPW // _CH


def _sc_body(uids, iids, utab, itab, out, uidx, iidx, urows, irows, outv,
             usem, isem):
    wid = lax.axis_index("s") * _NC + lax.axis_index("c")
    base = wid * _BPW

    pltpu.sync_copy(uids.at[pl.ds(base, _BPW)], uidx)
    pltpu.sync_copy(iids.at[pl.ds(base, _BPW)], iidx)

    one = jnp.ones((_L,), jnp.int32)
    for i in range(_BPW // _L):
        sl = pl.ds(i * _L, _L)
        uidx[sl] = uidx[sl] + one
        iidx[sl] = iidx[sl] + one

    lanes = lax.iota(jnp.int32, _L)
    row64 = pl.ds(0, EMBED_DIM)

    for c in range(_NCH):
        def fire(g, _):
            b = g * _L
            uvec = uidx[pl.ds(c * _CH + b, _L)]
            ivec = iidx[pl.ds(c * _CH + b, _L)]
            for l in range(_L):
                pltpu.async_copy(utab.at[uvec[l]],
                                 urows.at[b + l, row64], usem)
                pltpu.async_copy(itab.at[ivec[l]],
                                 irows.at[b + l, row64], isem)
            return 0

        lax.fori_loop(0, _CH // _L, fire, 0)

        def drain(k, _):
            pltpu.make_async_copy(utab.at[0], urows.at[k, row64],
                                  usem).wait()
            pltpu.make_async_copy(itab.at[0], irows.at[k, row64],
                                  isem).wait()
            return 0

        lax.fori_loop(0, _CH, drain, 0)

        def group(g, _):
            rows = g * _L + lanes

            def dstep(d, acc):
                col = jnp.full((_L,), d, jnp.int32)
                u = plsc.load_gather(urows, [rows, col])
                v = plsc.load_gather(irows, [rows, col])
                return acc + u * v

            acc = lax.fori_loop(0, EMBED_DIM, dstep,
                                jnp.zeros((_L,), jnp.float32))
            outv[pl.ds(c * _CH + g * _L, _L)] = acc
            return 0

        lax.fori_loop(0, _CH // _L, group, 0)

    pltpu.sync_copy(outv, out.at[pl.ds(base, _BPW)])


@jax.jit
def kernel(user_ids, item_ids, user_table, item_table):
    mesh = plsc.VectorSubcoreMesh(
        core_axis_name="c", subcore_axis_name="s",
        num_cores=_NC, num_subcores=_NS)
    run = pl.kernel(
        _sc_body,
        out_type=jax.ShapeDtypeStruct((BATCH,), jnp.float32),
        mesh=mesh,
        scratch_types=[
            pltpu.VMEM((_BPW,), jnp.int32),
            pltpu.VMEM((_BPW,), jnp.int32),
            pltpu.VMEM((_CH, 2 * EMBED_DIM), jnp.float32),
            pltpu.VMEM((_CH, 2 * EMBED_DIM), jnp.float32),
            pltpu.VMEM((_BPW,), jnp.float32),
            pltpu.SemaphoreType.DMA,
            pltpu.SemaphoreType.DMA,
        ],
        compiler_params=pltpu.CompilerParams(needs_layout_passes=False),
    )
    return run(user_ids, item_ids, user_table, item_table)


# speedup vs baseline: 2.7471x; 1.7599x over previous
import functools
import jax
import jax.numpy as jnp
from jax import lax
from jax.experimental import pallas as pl
from jax.experimental.pallas import tpu as pltpu
from jax.experimental.pallas import tpu_sc as plsc

BATCH = 16384
EMBED_DIM = 64
ROWS = 1000001

_NC = 2
_NS = 16
_L = 16
_BPT = BATCH // _NS
_DPC = EMBED_DIM // _NC

_TILES = ROWS // 128
_TPT = _TILES // _NS
_CPW = _TPT * 128
_REM0 = _NS * _CPW
_REM1 = _TILES * 128
_ROWBUF = 1000064


def _sc_body(uids, iids, utab, itab, utail, itail, out, uidx, iidx, gu, gi,
             acc, srow_u, srow_i, sem, gsem):
    cid = lax.axis_index("c")
    sid = lax.axis_index("s")
    base = pl.multiple_of(sid * _BPT, _BPT)

    pltpu.sync_copy(uids.at[pl.ds(base, _BPT)], uidx)
    pltpu.sync_copy(iids.at[pl.ds(base, _BPT)], iidx)
    one = jnp.ones((_L,), jnp.int32)

    def init(i, _):
        sl = pl.ds(pl.multiple_of(i * _L, _L), _L)
        uidx[sl] = uidx[sl] + one
        iidx[sl] = iidx[sl] + one
        acc[sl] = jnp.zeros((_L,), jnp.float32)
        return 0

    lax.fori_loop(0, _BPT // _L, init, 0)

    def per_d(d, _):
        dd = cid * _DPC + d
        c0 = pl.multiple_of(sid * _CPW, 128)
        cs = pltpu.async_copy(utab.at[dd, pl.ds(c0, _CPW)],
                              srow_u.at[pl.ds(c0, _CPW)], sem)
        ci = pltpu.async_copy(itab.at[dd, pl.ds(c0, _CPW)],
                              srow_i.at[pl.ds(c0, _CPW)], sem)

        @pl.when(sid == 0)
        def _tail():
            pltpu.async_copy(utab.at[dd, pl.ds(_REM0, _REM1 - _REM0)],
                             srow_u.at[pl.ds(_REM0, _REM1 - _REM0)], sem)
            pltpu.async_copy(itab.at[dd, pl.ds(_REM0, _REM1 - _REM0)],
                             srow_i.at[pl.ds(_REM0, _REM1 - _REM0)], sem)
            pltpu.async_copy(utail.at[dd], srow_u.at[pl.ds(_REM1, 128)],
                             sem)
            pltpu.async_copy(itail.at[dd], srow_i.at[pl.ds(_REM1, 128)],
                             sem)

        cs.wait()
        ci.wait()

        @pl.when(sid == 0)
        def _tail_wait():
            pltpu.make_async_copy(
                utab.at[dd, pl.ds(_REM0, _REM1 - _REM0)],
                srow_u.at[pl.ds(_REM0, _REM1 - _REM0)], sem).wait()
            pltpu.make_async_copy(
                itab.at[dd, pl.ds(_REM0, _REM1 - _REM0)],
                srow_i.at[pl.ds(_REM0, _REM1 - _REM0)], sem).wait()
            pltpu.make_async_copy(
                utail.at[dd], srow_u.at[pl.ds(_REM1, 128)], sem).wait()
            pltpu.make_async_copy(
                itail.at[dd], srow_i.at[pl.ds(_REM1, 128)], sem).wait()

        plsc.subcore_barrier()

        for j in range(_BPT // 128):
            sl = pl.ds(j * 128, 128)
            pltpu.async_copy(srow_u.at[uidx.at[pl.ds(j * 128, 128)]],
                             gu.at[sl], gsem)
            pltpu.async_copy(srow_i.at[iidx.at[pl.ds(j * 128, 128)]],
                             gi.at[sl], gsem)
        for j in range(_BPT // 128):
            sl = pl.ds(j * 128, 128)
            pltpu.make_async_copy(srow_u.at[pl.ds(0, 128)], gu.at[sl],
                                  gsem).wait()
            pltpu.make_async_copy(srow_i.at[pl.ds(0, 128)], gi.at[sl],
                                  gsem).wait()

        def fma(i, _):
            sl = pl.ds(pl.multiple_of(i * _L, _L), _L)
            acc[sl] = acc[sl] + gu[sl] * gi[sl]
            return 0

        lax.fori_loop(0, _BPT // _L, fma, 0)

        plsc.subcore_barrier()
        return 0

    lax.fori_loop(0, _DPC, per_d, 0)

    pltpu.sync_copy(acc, out.at[cid, pl.ds(base, _BPT)])


def _add_body(a_ref, o_ref):
    o_ref[...] = a_ref[0, :] + a_ref[1, :]


@jax.jit
def kernel(user_ids, item_ids, user_table, item_table):
    mesh = plsc.VectorSubcoreMesh(
        core_axis_name="c", subcore_axis_name="s",
        num_cores=_NC, num_subcores=_NS)
    run = pl.kernel(
        _sc_body,
        out_type=jax.ShapeDtypeStruct((_NC, BATCH), jnp.float32),
        mesh=mesh,
        scratch_types=[
            pltpu.VMEM((_BPT,), jnp.int32),
            pltpu.VMEM((_BPT,), jnp.int32),
            pltpu.VMEM((_BPT,), jnp.float32),
            pltpu.VMEM((_BPT,), jnp.float32),
            pltpu.VMEM((_BPT,), jnp.float32),
            pltpu.VMEM_SHARED((_ROWBUF,), jnp.float32),
            pltpu.VMEM_SHARED((_ROWBUF,), jnp.float32),
            pltpu.SemaphoreType.DMA,
            pltpu.SemaphoreType.DMA,
        ],
        compiler_params=pltpu.CompilerParams(needs_layout_passes=False),
    )
    utail = jnp.pad(user_table[_REM1:], ((0, 128 - (ROWS - _REM1)), (0, 0))).T
    itail = jnp.pad(item_table[_REM1:], ((0, 128 - (ROWS - _REM1)), (0, 0))).T
    partial = run(user_ids, item_ids, user_table.T, item_table.T,
                  utail, itail)
    return pl.pallas_call(
        _add_body,
        out_shape=jax.ShapeDtypeStruct((BATCH,), jnp.float32),
    )(partial)


# speedup vs baseline: 3.0277x; 1.1022x over previous
import functools
import jax
import jax.numpy as jnp
from jax import lax
from jax.experimental import pallas as pl
from jax.experimental.pallas import tpu as pltpu
from jax.experimental.pallas import tpu_sc as plsc

BATCH = 16384
EMBED_DIM = 64
ROWS = 1000001

_NC = 2
_NS = 16
_L = 16
_BPT = BATCH // _NS
_DPC = EMBED_DIM // _NC

_TILES = ROWS // 128
_TPT = _TILES // _NS
_CPW = _TPT * 128
_REM0 = _NS * _CPW
_REM1 = _TILES * 128
_ROWBUF = 1000064


def _sc_body(uids, iids, utab, itab, utail, itail, out, uidx, iidx, gu, gi,
             acc, srow_u, srow_i, usem, isem, gsem):
    cid = lax.axis_index("c")
    sid = lax.axis_index("s")
    base = pl.multiple_of(sid * _BPT, _BPT)

    pltpu.sync_copy(uids.at[pl.ds(base, _BPT)], uidx)
    pltpu.sync_copy(iids.at[pl.ds(base, _BPT)], iidx)
    one = jnp.ones((_L,), jnp.int32)

    def init(i, _):
        sl = pl.ds(pl.multiple_of(i * _L, _L), _L)
        uidx[sl] = uidx[sl] + one
        iidx[sl] = iidx[sl] + one
        acc[sl] = jnp.zeros((_L,), jnp.float32)
        return 0

    lax.fori_loop(0, _BPT // _L, init, 0)

    c0 = pl.multiple_of(sid * _CPW, 128)

    def stage(tab, tail, srow, dd, s):
        pltpu.async_copy(tab.at[dd, pl.ds(c0, _CPW)],
                         srow.at[pl.ds(c0, _CPW)], s)

        @pl.when(sid == 0)
        def _tail():
            pltpu.async_copy(tab.at[dd, pl.ds(_REM0, _REM1 - _REM0)],
                             srow.at[pl.ds(_REM0, _REM1 - _REM0)], s)
            pltpu.async_copy(tail.at[dd], srow.at[pl.ds(_REM1, 128)], s)

    def stage_wait(tab, tail, srow, dd, s):
        pltpu.make_async_copy(tab.at[dd, pl.ds(c0, _CPW)],
                              srow.at[pl.ds(c0, _CPW)], s).wait()

        @pl.when(sid == 0)
        def _tail_wait():
            pltpu.make_async_copy(
                tab.at[dd, pl.ds(_REM0, _REM1 - _REM0)],
                srow.at[pl.ds(_REM0, _REM1 - _REM0)], s).wait()
            pltpu.make_async_copy(
                tail.at[dd], srow.at[pl.ds(_REM1, 128)], s).wait()

    def gather(srow, idx, dst):
        for j in range(_BPT // 128):
            pltpu.async_copy(srow.at[idx.at[pl.ds(j * 128, 128)]],
                             dst.at[pl.ds(j * 128, 128)], gsem)
        pltpu.make_async_copy(srow.at[pl.ds(0, _BPT)], dst, gsem).wait()

    stage(utab, utail, srow_u, cid * _DPC, usem)
    stage_wait(utab, utail, srow_u, cid * _DPC, usem)
    plsc.subcore_barrier()

    def per_d(d, _):
        dd = cid * _DPC + d
        stage(itab, itail, srow_i, dd, isem)
        gather(srow_u, uidx, gu)
        stage_wait(itab, itail, srow_i, dd, isem)
        plsc.subcore_barrier()

        @pl.when(d + 1 < _DPC)
        def _pf():
            stage(utab, utail, srow_u, dd + 1, usem)

        gather(srow_i, iidx, gi)

        def fma(i, _):
            sl = pl.ds(pl.multiple_of(i * _L, _L), _L)
            acc[sl] = acc[sl] + gu[sl] * gi[sl]
            return 0

        lax.fori_loop(0, _BPT // _L, fma, 0)

        @pl.when(d + 1 < _DPC)
        def _pf_wait():
            stage_wait(utab, utail, srow_u, dd + 1, usem)

        plsc.subcore_barrier()
        return 0

    lax.fori_loop(0, _DPC, per_d, 0)

    pltpu.sync_copy(acc, out.at[cid, pl.ds(base, _BPT)])


def _add_body(a_ref, o_ref):
    o_ref[...] = a_ref[0, :] + a_ref[1, :]


@jax.jit
def kernel(user_ids, item_ids, user_table, item_table):
    mesh = plsc.VectorSubcoreMesh(
        core_axis_name="c", subcore_axis_name="s",
        num_cores=_NC, num_subcores=_NS)
    run = pl.kernel(
        _sc_body,
        out_type=jax.ShapeDtypeStruct((_NC, BATCH), jnp.float32),
        mesh=mesh,
        scratch_types=[
            pltpu.VMEM((_BPT,), jnp.int32),
            pltpu.VMEM((_BPT,), jnp.int32),
            pltpu.VMEM((_BPT,), jnp.float32),
            pltpu.VMEM((_BPT,), jnp.float32),
            pltpu.VMEM((_BPT,), jnp.float32),
            pltpu.VMEM_SHARED((_ROWBUF,), jnp.float32),
            pltpu.VMEM_SHARED((_ROWBUF,), jnp.float32),
            pltpu.SemaphoreType.DMA,
            pltpu.SemaphoreType.DMA,
            pltpu.SemaphoreType.DMA,
        ],
        compiler_params=pltpu.CompilerParams(needs_layout_passes=False),
    )
    utail = jnp.pad(user_table[_REM1:], ((0, 128 - (ROWS - _REM1)), (0, 0))).T
    itail = jnp.pad(item_table[_REM1:], ((0, 128 - (ROWS - _REM1)), (0, 0))).T
    partial = run(user_ids, item_ids, user_table.T, item_table.T,
                  utail, itail)
    return pl.pallas_call(
        _add_body,
        out_shape=jax.ShapeDtypeStruct((BATCH,), jnp.float32),
    )(partial)
